# baseline (speedup 1.0000x reference)
import functools

import jax
import jax.numpy as jnp
from jax import lax
from jax.experimental import pallas as pl
from jax.experimental.pallas import tpu as pltpu
from jax.experimental.pallas import tpu_sc as plsc

EMBED = 64
LN_EPS = 1e-12

_NC = 2
_NS = 16
_NW = _NC * _NS
_LANES = 16
_Q = EMBED // _LANES

_B_BLK = 1024


def _rsqrt_scalar(v):
    i = lax.bitcast_convert_type(v, jnp.int32)
    i = jnp.int32(0x5F3759DF) - lax.shift_right_logical(i, 1)
    y = lax.bitcast_convert_type(i, jnp.float32)
    for _ in range(3):
        y = y * (1.5 - 0.5 * v * y * y)
    return y


def _make_sc_kernel(n_rows):
    rows_per_worker = n_rows // _NW
    n_blk = rows_per_worker // _B_BLK
    mesh = plsc.VectorSubcoreMesh(core_axis_name="c", subcore_axis_name="s")

    @functools.partial(
        pl.kernel,
        mesh=mesh,
        out_type=jax.ShapeDtypeStruct((n_rows, EMBED), jnp.float32),
        scratch_types=[
            pltpu.VMEM((_B_BLK,), jnp.int32),
            pltpu.VMEM((_B_BLK, EMBED), jnp.float32),
            pltpu.VMEM((EMBED,), jnp.float32),
            pltpu.VMEM((EMBED,), jnp.float32),
            pltpu.SemaphoreType.DMA,
        ],
        compiler_params=pltpu.CompilerParams(
            needs_layout_passes=False, use_tc_tiling_on_sc=False
        ),
    )
    def sc_kernel(idx_hbm, table_hbm, gamma_hbm, beta_hbm, out_hbm,
                  idx_v, rows_v, gamma_v, beta_v, sem):
        wid = lax.axis_index("s") * _NC + lax.axis_index("c")
        base = wid * rows_per_worker

        pltpu.sync_copy(gamma_hbm, gamma_v)
        pltpu.sync_copy(beta_hbm, beta_v)

        def block_body(blk, _):
            row0 = base + blk * _B_BLK
            pltpu.sync_copy(idx_hbm.at[pl.ds(row0, _B_BLK)], idx_v)
            pltpu.async_copy(table_hbm.at[idx_v], rows_v, sem).wait()

            g = [gamma_v[pl.ds(q * _LANES, _LANES)] for q in range(_Q)]
            b = [beta_v[pl.ds(q * _LANES, _LANES)] for q in range(_Q)]

            def row_body(r, _):
                x = [rows_v[r, pl.ds(q * _LANES, _LANES)] for q in range(_Q)]
                t = (x[0] + x[1]) + (x[2] + x[3])
                t2 = (x[0] * x[0] + x[1] * x[1]) + (x[2] * x[2] + x[3] * x[3])
                s = jnp.sum(t)
                ss = jnp.sum(t2)
                mean = s * (1.0 / EMBED)
                var = ss * (1.0 / EMBED) - mean * mean
                inv = _rsqrt_scalar(var + LN_EPS)
                for q in range(_Q):
                    yv = (x[q] - mean) * inv * g[q] + b[q]
                    rows_v[r, pl.ds(q * _LANES, _LANES)] = yv
                return 0

            lax.fori_loop(0, _B_BLK, row_body, 0, unroll=4)
            pltpu.sync_copy(rows_v, out_hbm.at[pl.ds(row0, _B_BLK)])
            return 0

        lax.fori_loop(0, n_blk, block_body, 0)

    return sc_kernel


def kernel(x, table, gamma, beta):
    batch, seqlen = x.shape
    n_rows = batch * seqlen
    x_flat = x.reshape(n_rows)
    out = _make_sc_kernel(n_rows)(x_flat, table, gamma, beta)
    return out.reshape(batch, seqlen, EMBED)
